# baseline (speedup 1.0000x reference)
import functools

import jax
import jax.numpy as jnp
from jax import lax
from jax.experimental import pallas as pl
from jax.experimental.pallas import tpu as pltpu
from jax.experimental.pallas import tpu_sc as plsc

PRE = 128
D = 18432
B = 2048
NW = 32
W_LO = 512
W_HI = 640
NG = B // 16

_mesh = plsc.VectorSubcoreMesh(core_axis_name="c", subcore_axis_name="s")


@functools.partial(
    pl.kernel,
    mesh=_mesh,
    out_type=jax.ShapeDtypeStruct((B, D), jnp.float32),
    scratch_types=[
        pltpu.VMEM((B,), jnp.int32),
        pltpu.VMEM((PRE, W_HI), jnp.float32),
        pltpu.SemaphoreType.DMA,
    ],
)
def _gather_kernel(idx_hbm, table_hbm, out_hbm, idx_v, table_v, sem):
    wid = lax.axis_index("s") * 2 + lax.axis_index("c")

    def run(width, col0):
        col0 = pl.multiple_of(col0, 128)
        pltpu.async_copy(idx_hbm, idx_v, sem)
        pltpu.async_copy(
            table_hbm.at[:, pl.ds(col0, width)], table_v.at[:, pl.ds(0, width)], sem
        )
        pltpu.make_async_copy(idx_hbm, idx_v, sem).wait()
        pltpu.make_async_copy(
            table_hbm.at[:, pl.ds(col0, width)], table_v.at[:, pl.ds(0, width)], sem
        ).wait()

        def body(g, carry):
            idx16 = idx_v[pl.ds(g * 16, 16)]
            for j in range(16):
                row = idx16[j]
                pltpu.async_copy(
                    table_v.at[row, pl.ds(0, width)],
                    out_hbm.at[g * 16 + j, pl.ds(col0, width)],
                    sem,
                )

            @pl.when(g > 1)
            def _():
                for j in range(16):
                    pltpu.make_async_copy(
                        table_v.at[0, pl.ds(0, width)],
                        out_hbm.at[0, pl.ds(col0, width)],
                        sem,
                    ).wait()

            return carry

        lax.fori_loop(0, NG, body, 0)
        for j in range(32):
            pltpu.make_async_copy(
                table_v.at[0, pl.ds(0, width)],
                out_hbm.at[0, pl.ds(col0, width)],
                sem,
            ).wait()

    @pl.when(wid < 16)
    def _():
        run(W_LO, wid * W_LO)

    @pl.when(wid >= 16)
    def _():
        run(W_HI, 16 * W_LO + (wid - 16) * W_HI)


def kernel(prefix, embedding):
    idx = prefix.reshape(B)
    out = _gather_kernel(idx, embedding)
    return out.reshape(prefix.shape[0], prefix.shape[1], D)


# speedup vs baseline: 1.0097x; 1.0097x over previous
import functools

import jax
import jax.numpy as jnp
from jax import lax
from jax.experimental import pallas as pl
from jax.experimental.pallas import tpu as pltpu
from jax.experimental.pallas import tpu_sc as plsc

PRE = 128
D = 18432
B = 2048
NW = 32
W_LO = 512
W_HI = 640
NG = B // 16

_mesh = plsc.VectorSubcoreMesh(core_axis_name="c", subcore_axis_name="s")


@functools.partial(
    pl.kernel,
    mesh=_mesh,
    out_type=jax.ShapeDtypeStruct((B, D), jnp.float32),
    scratch_types=[
        pltpu.VMEM((B,), jnp.int32),
        pltpu.VMEM((PRE, W_HI), jnp.float32),
        pltpu.SemaphoreType.DMA,
    ],
)
def _gather_kernel(idx_hbm, table_hbm, out_hbm, idx_v, table_v, sem):
    wid = lax.axis_index("s") * 2 + lax.axis_index("c")

    def run(width, col0):
        col0 = pl.multiple_of(col0, 128)
        pltpu.async_copy(idx_hbm, idx_v, sem)
        pltpu.async_copy(
            table_hbm.at[:, pl.ds(col0, width)], table_v.at[:, pl.ds(0, width)], sem
        )
        pltpu.make_async_copy(idx_hbm, idx_v, sem).wait()
        pltpu.make_async_copy(
            table_hbm.at[:, pl.ds(col0, width)], table_v.at[:, pl.ds(0, width)], sem
        ).wait()

        def body(g, carry):
            idx16 = idx_v[pl.ds(g * 16, 16)]
            for j in range(16):
                row = idx16[j]
                pltpu.async_copy(
                    table_v.at[row, pl.ds(0, width)],
                    out_hbm.at[g * 16 + j, pl.ds(col0, width)],
                    sem,
                )

            @pl.when(g > 1)
            def _():
                pltpu.make_async_copy(
                    table_v.at[pl.ds(0, 16), pl.ds(0, width)],
                    out_hbm.at[pl.ds(0, 16), pl.ds(col0, width)],
                    sem,
                ).wait()

            return carry

        lax.fori_loop(0, NG, body, 0)
        for j in range(2):
            pltpu.make_async_copy(
                table_v.at[pl.ds(0, 16), pl.ds(0, width)],
                out_hbm.at[pl.ds(0, 16), pl.ds(col0, width)],
                sem,
            ).wait()

    @pl.when(wid < 16)
    def _():
        run(W_LO, wid * W_LO)

    @pl.when(wid >= 16)
    def _():
        run(W_HI, 16 * W_LO + (wid - 16) * W_HI)


def kernel(prefix, embedding):
    idx = prefix.reshape(B)
    out = _gather_kernel(idx, embedding)
    return out.reshape(prefix.shape[0], prefix.shape[1], D)


# speedup vs baseline: 1.0426x; 1.0326x over previous
import functools

import jax
import jax.numpy as jnp
from jax import lax
from jax.experimental import pallas as pl
from jax.experimental.pallas import tpu as pltpu
from jax.experimental.pallas import tpu_sc as plsc

PRE = 128
D = 18432
B = 2048
NW = 32
GCOL = 1152
W_CACHE = 640
NG = B // 16

_mesh = plsc.VectorSubcoreMesh(core_axis_name="c", subcore_axis_name="s")


@functools.partial(
    pl.kernel,
    mesh=_mesh,
    out_type=jax.ShapeDtypeStruct((B, D), jnp.float32),
    scratch_types=[
        pltpu.VMEM((B,), jnp.int32),
        pltpu.VMEM((PRE, W_CACHE), jnp.float32),
        pltpu.SemaphoreType.DMA,
    ],
)
def _gather_kernel(idx_hbm, table_hbm, out_hbm, idx_v, table_v, sem):
    wid = lax.axis_index("s") * 2 + lax.axis_index("c")
    k = wid // 2
    p = wid % 2
    cache_col = pl.multiple_of(k * GCOL + p * 512, 128)

    pltpu.async_copy(idx_hbm, idx_v, sem)
    pltpu.async_copy(table_hbm.at[:, pl.ds(cache_col, W_CACHE)], table_v, sem)
    pltpu.make_async_copy(idx_hbm, idx_v, sem).wait()
    pltpu.make_async_copy(
        table_hbm.at[:, pl.ds(cache_col, W_CACHE)], table_v, sem
    ).wait()

    def run_phase(g0, n_groups, width, colg, colv):
        colg = pl.multiple_of(colg, 128)

        def body(g, carry):
            base = (g0 + g) * 16
            idx16 = idx_v[pl.ds(base, 16)]
            for j in range(16):
                row = idx16[j]
                pltpu.async_copy(
                    table_v.at[row, pl.ds(colv, width)],
                    out_hbm.at[base + j, pl.ds(colg, width)],
                    sem,
                )

            @pl.when(g > 1)
            def _():
                pltpu.make_async_copy(
                    table_v.at[pl.ds(0, 16), pl.ds(colv, width)],
                    out_hbm.at[pl.ds(0, 16), pl.ds(colg, width)],
                    sem,
                ).wait()

            return carry

        lax.fori_loop(0, n_groups, body, 0)
        for _ in range(2):
            pltpu.make_async_copy(
                table_v.at[pl.ds(0, 16), pl.ds(colv, width)],
                out_hbm.at[pl.ds(0, 16), pl.ds(colg, width)],
                sem,
            ).wait()

    @pl.when(p == 0)
    def _():
        run_phase(0, NG // 2, W_CACHE, k * GCOL, 0)
        run_phase(NG // 2, NG // 2, 512, k * GCOL, 0)

    @pl.when(p == 1)
    def _():
        run_phase(0, NG // 2, 512, k * GCOL + W_CACHE, 128)
        run_phase(NG // 2, NG // 2, W_CACHE, k * GCOL + 512, 0)


def kernel(prefix, embedding):
    idx = prefix.reshape(B)
    out = _gather_kernel(idx, embedding)
    return out.reshape(prefix.shape[0], prefix.shape[1], D)
